# baseline (speedup 1.0000x reference)
import functools

import jax
import jax.numpy as jnp
from jax import lax
from jax.experimental import pallas as pl
from jax.experimental.pallas import tpu as pltpu
from jax.experimental.pallas import tpu_sc as plsc

_B = 1024
_L = 200
_D = 64
_SPLITS = ((0, 104), (104, 96))
_NC, _NS = 2, 16
_NW = _NC * _NS
_RPW = _B // _NW
_G = 2
_LANES = 16


@functools.partial(
    pl.kernel,
    out_type=jax.ShapeDtypeStruct((_B, _L, _D), jnp.float32),
    mesh=plsc.VectorSubcoreMesh(core_axis_name="c", subcore_axis_name="s",
                                num_cores=_NC),
    scratch_types=(
        [pltpu.VMEM((_L,), jnp.int32)] * _G
        + [pltpu.VMEM((_G, _L, 2 * _D), jnp.float32),
         pltpu.VMEM((_G, _L, _D), jnp.float32),
         pltpu.VMEM((_L // 2, 2 * _D), jnp.float32)]
        + [pltpu.SemaphoreType.DMA] * (3 * _G)
    ),
)
def _embed_kernel(ids_hbm, t128_hbm, wpe2_hbm, out_hbm, *refs):
    idx_v = refs[0:_G]
    rows_v, sum_v, wpe2_v = refs[_G:_G + 3]
    sems = refs[_G + 3:]
    idx_sem = sems[0:_G]
    g_sem = sems[_G:2 * _G]
    out_sem = sems[2 * _G:3 * _G]

    wid = lax.axis_index("s") * _NC + lax.axis_index("c")
    base = wid * _RPW

    pltpu.sync_copy(wpe2_hbm, wpe2_v)

    @pl.loop(0, _RPW, step=_G)
    def _group(g0):
        ins = []
        for r in range(_G):
            row = base + g0 + r
            ins.append(pltpu.async_copy(
                ids_hbm.at[pl.ds(row * _L, _L)], idx_v[r], idx_sem[r]))

        gaths = []
        for r in range(_G):
            ins[r].wait()
            descs = []
            for off, size in _SPLITS:
                descs.append(pltpu.async_copy(
                    t128_hbm.at[idx_v[r].at[pl.ds(off, size)]],
                    rows_v.at[r, pl.ds(off, size)], g_sem[r]))
            gaths.append(descs)

        outs = []
        for r in range(_G):
            for d in gaths[r]:
                d.wait()

            @pl.loop(0, _L // 2, unroll=4)
            def _add(l2):
                for h in range(2):
                    for c in range(_D // _LANES):
                        sl = pl.ds(c * _LANES, _LANES)
                        wsl = pl.ds(h * _D + c * _LANES, _LANES)
                        sum_v[r, 2 * l2 + h, sl] = (
                            rows_v[r, 2 * l2 + h, sl] + wpe2_v[l2, wsl])

            outs.append(pltpu.async_copy(sum_v.at[r],
                                         out_hbm.at[base + g0 + r],
                                         out_sem[r]))

        for d in outs:
            d.wait()


def kernel(input_ids, wte_table, wpe_table):
    t128 = jnp.pad(wte_table, ((0, 0), (0, _D)))
    ids = input_ids.reshape(-1).astype(jnp.int32)
    wpe2 = wpe_table[:_L].reshape(_L // 2, 2 * _D)
    return _embed_kernel(ids, t128, wpe2)
